# Initial kernel scaffold; baseline (speedup 1.0000x reference)
import jax
import jax.numpy as jnp
from jax.experimental import pallas as pl


def kernel(init_pos_ids, hop_dis_ids, time_dis_ids, pos_table, hop_table, time_table, ln_weight, ln_bias):
    raise NotImplementedError("write your pallas kernel here")


# speedup vs baseline: 3.3326x; 3.3326x over previous
import functools

import jax
import jax.numpy as jnp
import numpy as np
from jax import lax
from jax.experimental import pallas as pl
from jax.experimental.pallas import tpu as pltpu
from jax.experimental.pallas import tpu_sc as plsc

B, L = 1024, 200
H = 64
N = B * L
EPS = 1e-12

NC, NS, LANES = 2, 16, 16
NW = NC * NS
ROWS_PER_W = N // NW
CHUNK = 128
NCHUNKS = ROWS_PER_W // CHUNK

_RSQRT_MAGIC = np.int32(0x5F3759DF)
_GDN = lax.GatherDimensionNumbers(
    offset_dims=(), collapsed_slice_dims=(0,), start_index_map=(0,))


def _lane_shuffle(t, idx):
    return lax.gather(t, idx.reshape(16, 1), _GDN, slice_sizes=(1,),
                      mode=lax.GatherScatterMode.PROMISE_IN_BOUNDS)


def _allsum(t):
    lanes = lax.iota(jnp.int32, 16)
    for k in (8, 4, 2, 1):
        t = t + _lane_shuffle(t, lanes ^ np.int32(k))
    return t


def _rsqrt_vec(x):
    i = plsc.bitcast(x, jnp.int32)
    i = _RSQRT_MAGIC - lax.shift_right_logical(i, 1)
    y = plsc.bitcast(i, jnp.float32)
    for _ in range(3):
        y = y * (1.5 - 0.5 * x * y * y)
    return y


def _body(pos_ids, hop_ids, time_ids, pos_tbl, hop_tbl, lnw_hbm, lnb_hbm,
          out_hbm, pidx, hidx, tidx, prow, hrow, trow, wvec, bvec, sem):
    wid = lax.axis_index("s") * NC + lax.axis_index("c")
    base = wid * ROWS_PER_W
    pltpu.sync_copy(lnw_hbm, wvec)
    pltpu.sync_copy(lnb_hbm, bvec)

    ws = [wvec[pl.ds(16 * j, 16)] for j in range(4)]
    bs = [bvec[pl.ds(16 * j, 16)] for j in range(4)]

    def chunk_body(g, _):
        off = pl.multiple_of(base + g * CHUNK, CHUNK)
        pltpu.sync_copy(pos_ids.at[pl.ds(off, CHUNK)], pidx)
        pltpu.sync_copy(hop_ids.at[pl.ds(off, CHUNK)], hidx)
        pltpu.sync_copy(time_ids.at[pl.ds(off, CHUNK)], tidx)
        cp = pltpu.async_copy(pos_tbl.at[pidx], prow, sem)
        ch = pltpu.async_copy(hop_tbl.at[hidx], hrow, sem)
        ct = pltpu.async_copy(hop_tbl.at[tidx], trow, sem)
        cp.wait()
        ch.wait()
        ct.wait()

        def row_body(r, _):
            s = [prow[r, pl.ds(16 * j, 16)] + hrow[r, pl.ds(16 * j, 16)]
                 + trow[r, pl.ds(16 * j, 16)] for j in range(4)]
            tot = (s[0] + s[1]) + (s[2] + s[3])
            sq = (s[0] * s[0] + s[1] * s[1]) + (s[2] * s[2] + s[3] * s[3])
            mean = _allsum(tot) * (1.0 / H)
            msq = _allsum(sq) * (1.0 / H)
            var = msq - mean * mean
            rstd = _rsqrt_vec(var + EPS)
            for j in range(4):
                prow[r, pl.ds(16 * j, 16)] = (s[j] - mean) * rstd * ws[j] + bs[j]
            return 0

        lax.fori_loop(0, CHUNK, row_body, 0, unroll=2)
        pltpu.sync_copy(prow, out_hbm.at[pl.ds(off, CHUNK)])
        return 0

    lax.fori_loop(0, NCHUNKS, chunk_body, 0)


@functools.partial(jax.jit, static_argnames=())
def _run(pos_ids, hop_ids, time_ids, pos_tbl, hop_tbl, lnw, lnb):
    mesh = plsc.VectorSubcoreMesh(core_axis_name="c", subcore_axis_name="s",
                                  num_cores=NC, num_subcores=NS)
    f = pl.kernel(
        _body,
        out_type=jax.ShapeDtypeStruct((N, H), jnp.float32),
        mesh=mesh,
        compiler_params=pltpu.CompilerParams(needs_layout_passes=False,
                                             use_tc_tiling_on_sc=False),
        scratch_types=[
            pltpu.VMEM((CHUNK,), jnp.int32),
            pltpu.VMEM((CHUNK,), jnp.int32),
            pltpu.VMEM((CHUNK,), jnp.int32),
            pltpu.VMEM((CHUNK, H), jnp.float32),
            pltpu.VMEM((CHUNK, H), jnp.float32),
            pltpu.VMEM((CHUNK, H), jnp.float32),
            pltpu.VMEM((H,), jnp.float32),
            pltpu.VMEM((H,), jnp.float32),
            pltpu.SemaphoreType.DMA,
        ],
    )
    return f(pos_ids, hop_ids, time_ids, pos_tbl, hop_tbl, lnw, lnb)


def kernel(init_pos_ids, hop_dis_ids, time_dis_ids, pos_table, hop_table,
           time_table, ln_weight, ln_bias):
    del time_table
    pos_ids = init_pos_ids.reshape(N).astype(jnp.int32)
    hop_ids = hop_dis_ids.reshape(N).astype(jnp.int32)
    time_ids = time_dis_ids.reshape(N).astype(jnp.int32)
    out = _run(pos_ids, hop_ids, time_ids, pos_table, hop_table,
               ln_weight, ln_bias)
    return out.reshape(B, L, H)


# speedup vs baseline: 4.3321x; 1.2999x over previous
import functools

import jax
import jax.numpy as jnp
import numpy as np
from jax import lax
from jax.experimental import pallas as pl
from jax.experimental.pallas import tpu as pltpu
from jax.experimental.pallas import tpu_sc as plsc

B, L = 1024, 200
H = 64
N = B * L
EPS = 1e-12

NC, NS, LANES = 2, 16, 16
NW = NC * NS
ROWS_PER_W = N // NW
CHUNK = 128
NCHUNKS = ROWS_PER_W // CHUNK

_RSQRT_MAGIC = np.int32(0x5F3759DF)
_GDN = lax.GatherDimensionNumbers(
    offset_dims=(), collapsed_slice_dims=(0,), start_index_map=(0,))


def _lane_shuffle(t, idx):
    return lax.gather(t, idx.reshape(16, 1), _GDN, slice_sizes=(1,),
                      mode=lax.GatherScatterMode.PROMISE_IN_BOUNDS)


def _allsum(t):
    lanes = lax.iota(jnp.int32, 16)
    for k in (8, 4, 2, 1):
        t = t + _lane_shuffle(t, lanes ^ np.int32(k))
    return t


def _rsqrt_vec(x):
    i = plsc.bitcast(x, jnp.int32)
    i = _RSQRT_MAGIC - lax.shift_right_logical(i, 1)
    y = plsc.bitcast(i, jnp.float32)
    for _ in range(3):
        y = y * (1.5 - 0.5 * x * y * y)
    return y


def _body(ids3_hbm, pos_tbl, hop_tbl, lnw_hbm, lnb_hbm, out_hbm,
          idx0, idx1, prow0, prow1, hrow0, hrow1, trow0, trow1, obuf0, obuf1,
          wvec, bvec, semg0, semg1, semi0, semi1, semo0, semo1):
    wid = lax.axis_index("s") * NC + lax.axis_index("c")
    base = wid * ROWS_PER_W
    pltpu.sync_copy(lnw_hbm, wvec)
    pltpu.sync_copy(lnb_hbm, bvec)

    ws = [wvec[pl.ds(16 * j, 16)] for j in range(4)]
    bs = [bvec[pl.ds(16 * j, 16)] for j in range(4)]

    slots = ((idx0, prow0, hrow0, trow0, obuf0, semg0, semi0, semo0),
             (idx1, prow1, hrow1, trow1, obuf1, semg1, semi1, semo1))

    def off_of(g):
        return pl.multiple_of(base + g * CHUNK, CHUNK)

    def fire_gathers(slot, off_is_ready=True):
        idxb, prowb, hrowb, trowb = slot[0], slot[1], slot[2], slot[3]
        semg = slot[5]
        pltpu.async_copy(pos_tbl.at[idxb.at[0]], prowb, semg)
        pltpu.async_copy(hop_tbl.at[idxb.at[1]], hrowb, semg)
        pltpu.async_copy(hop_tbl.at[idxb.at[2]], trowb, semg)

    def wait_gathers(slot):
        prowb, hrowb, trowb, semg = slot[1], slot[2], slot[3], slot[5]
        pltpu.make_async_copy(pos_tbl.at[pl.ds(0, CHUNK)], prowb, semg).wait()
        pltpu.make_async_copy(hop_tbl.at[pl.ds(0, CHUNK)], hrowb, semg).wait()
        pltpu.make_async_copy(hop_tbl.at[pl.ds(0, CHUNK)], trowb, semg).wait()

    def compute_chunk(slot):
        prowb, hrowb, trowb, obufb = slot[1], slot[2], slot[3], slot[4]

        def row_body(r, _):
            s = [prowb[r, pl.ds(16 * j, 16)] + hrowb[r, pl.ds(16 * j, 16)]
                 + trowb[r, pl.ds(16 * j, 16)] for j in range(4)]
            tot = (s[0] + s[1]) + (s[2] + s[3])
            sq = (s[0] * s[0] + s[1] * s[1]) + (s[2] * s[2] + s[3] * s[3])
            mean = _allsum(tot) * (1.0 / H)
            msq = _allsum(sq) * (1.0 / H)
            var = msq - mean * mean
            rstd = _rsqrt_vec(var + EPS)
            for j in range(4):
                obufb[r, pl.ds(16 * j, 16)] = (s[j] - mean) * rstd * ws[j] + bs[j]
            return 0

        lax.fori_loop(0, CHUNK, row_body, 0, unroll=4)

    for b in (0, 1):
        pltpu.sync_copy(ids3_hbm.at[:, pl.ds(off_of(b), CHUNK)], slots[b][0])
        fire_gathers(slots[b])

    def pair_body(gg, _):
        for b in (0, 1):
            slot = slots[b]
            idxb, obufb, semi, semo = slot[0], slot[4], slot[6], slot[7]
            g = 2 * gg + b
            off_g = off_of(g)
            wait_gathers(slot)

            @pl.when(g + 2 < NCHUNKS)
            def _():
                pltpu.async_copy(
                    ids3_hbm.at[:, pl.ds(off_of(g + 2), CHUNK)], idxb, semi)

            @pl.when(g >= 2)
            def _():
                pltpu.make_async_copy(
                    obufb, out_hbm.at[pl.ds(0, CHUNK)], semo).wait()

            compute_chunk(slot)
            pltpu.async_copy(obufb, out_hbm.at[pl.ds(off_g, CHUNK)], semo)

            @pl.when(g + 2 < NCHUNKS)
            def _():
                pltpu.make_async_copy(
                    ids3_hbm.at[:, pl.ds(0, CHUNK)], idxb, semi).wait()
                fire_gathers(slot)

        return 0

    lax.fori_loop(0, NCHUNKS // 2, pair_body, 0)

    for b in (0, 1):
        pltpu.make_async_copy(
            slots[b][4], out_hbm.at[pl.ds(0, CHUNK)], slots[b][7]).wait()


@functools.partial(jax.jit, static_argnames=())
def _run(ids3, pos_tbl, hop_tbl, lnw, lnb):
    mesh = plsc.VectorSubcoreMesh(core_axis_name="c", subcore_axis_name="s",
                                  num_cores=NC, num_subcores=NS)
    f = pl.kernel(
        _body,
        out_type=jax.ShapeDtypeStruct((N, H), jnp.float32),
        mesh=mesh,
        compiler_params=pltpu.CompilerParams(needs_layout_passes=False,
                                             use_tc_tiling_on_sc=False),
        scratch_types=[
            pltpu.VMEM((3, CHUNK), jnp.int32),
            pltpu.VMEM((3, CHUNK), jnp.int32),
            pltpu.VMEM((CHUNK, H), jnp.float32),
            pltpu.VMEM((CHUNK, H), jnp.float32),
            pltpu.VMEM((CHUNK, H), jnp.float32),
            pltpu.VMEM((CHUNK, H), jnp.float32),
            pltpu.VMEM((CHUNK, H), jnp.float32),
            pltpu.VMEM((CHUNK, H), jnp.float32),
            pltpu.VMEM((CHUNK, H), jnp.float32),
            pltpu.VMEM((CHUNK, H), jnp.float32),
            pltpu.VMEM((H,), jnp.float32),
            pltpu.VMEM((H,), jnp.float32),
            pltpu.SemaphoreType.DMA,
            pltpu.SemaphoreType.DMA,
            pltpu.SemaphoreType.DMA,
            pltpu.SemaphoreType.DMA,
            pltpu.SemaphoreType.DMA,
            pltpu.SemaphoreType.DMA,
        ],
    )
    return f(ids3, pos_tbl, hop_tbl, lnw, lnb)


def kernel(init_pos_ids, hop_dis_ids, time_dis_ids, pos_table, hop_table,
           time_table, ln_weight, ln_bias):
    del time_table
    ids3 = jnp.stack([init_pos_ids.reshape(N).astype(jnp.int32),
                      hop_dis_ids.reshape(N).astype(jnp.int32),
                      time_dis_ids.reshape(N).astype(jnp.int32)])
    out = _run(ids3, pos_table, hop_table, ln_weight, ln_bias)
    return out.reshape(B, L, H)


# speedup vs baseline: 6.5490x; 1.5117x over previous
import functools

import jax
import jax.numpy as jnp
import numpy as np
from jax import lax
from jax.experimental import pallas as pl
from jax.experimental.pallas import tpu as pltpu
from jax.experimental.pallas import tpu_sc as plsc

B, L = 1024, 200
H = 64
N = B * L
EPS = 1e-12

NC, NS, LANES = 2, 16, 16
NW = NC * NS
ROWS_PER_W = N // NW
CHUNK = 128
NCHUNKS = ROWS_PER_W // CHUNK

_RSQRT_MAGIC = np.int32(0x5F3759DF)
_GDN = lax.GatherDimensionNumbers(
    offset_dims=(), collapsed_slice_dims=(0,), start_index_map=(0,))


def _lane_shuffle(t, idx):
    return lax.gather(t, idx.reshape(16, 1), _GDN, slice_sizes=(1,),
                      mode=lax.GatherScatterMode.PROMISE_IN_BOUNDS)


def _allsum(t):
    lanes = lax.iota(jnp.int32, 16)
    for k in (8, 4, 2, 1):
        t = t + _lane_shuffle(t, lanes ^ np.int32(k))
    return t


def _rsqrt_vec(x):
    i = plsc.bitcast(x, jnp.int32)
    i = _RSQRT_MAGIC - lax.shift_right_logical(i, 1)
    y = plsc.bitcast(i, jnp.float32)
    for _ in range(3):
        y = y * (1.5 - 0.5 * x * y * y)
    return y


def _body(ids3_hbm, pos_tbl, hop_tbl, lnw_hbm, lnb_hbm, out_hbm,
          idx0, idx1, prow0, prow1, hrow0, hrow1, trow0, trow1, obuf0, obuf1,
          wvec, bvec, semg0, semg1, semi0, semi1, semo0, semo1):
    wid = lax.axis_index("s") * NC + lax.axis_index("c")
    base = wid * ROWS_PER_W
    pltpu.sync_copy(lnw_hbm, wvec)
    pltpu.sync_copy(lnb_hbm, bvec)

    ws = [wvec[pl.ds(16 * j, 16)] for j in range(4)]
    bs = [bvec[pl.ds(16 * j, 16)] for j in range(4)]

    slots = ((idx0, prow0, hrow0, trow0, obuf0, semg0, semi0, semo0),
             (idx1, prow1, hrow1, trow1, obuf1, semg1, semi1, semo1))

    def off_of(g):
        return pl.multiple_of(base + g * CHUNK, CHUNK)

    def fire_gathers(slot, off_is_ready=True):
        idxb, prowb, hrowb, trowb = slot[0], slot[1], slot[2], slot[3]
        semg = slot[5]
        pltpu.async_copy(pos_tbl.at[idxb.at[0]], prowb, semg)
        pltpu.async_copy(hop_tbl.at[idxb.at[1]], hrowb, semg)
        pltpu.async_copy(hop_tbl.at[idxb.at[2]], trowb, semg)

    def wait_gathers(slot):
        prowb, hrowb, trowb, semg = slot[1], slot[2], slot[3], slot[5]
        pltpu.make_async_copy(pos_tbl.at[pl.ds(0, CHUNK)], prowb, semg).wait()
        pltpu.make_async_copy(hop_tbl.at[pl.ds(0, CHUNK)], hrowb, semg).wait()
        pltpu.make_async_copy(hop_tbl.at[pl.ds(0, CHUNK)], trowb, semg).wait()

    def compute_chunk(slot):
        prowb, hrowb, trowb, obufb = slot[1], slot[2], slot[3], slot[4]
        lanes = lax.iota(jnp.int32, 16)
        G = 4

        def grp_body(it, _):
            rb = it * G
            s = [[prowb[rb + r, pl.ds(16 * j, 16)]
                  + hrowb[rb + r, pl.ds(16 * j, 16)]
                  + trowb[rb + r, pl.ds(16 * j, 16)] for j in range(4)]
                 for r in range(G)]
            tot = [(s[r][0] + s[r][1]) + (s[r][2] + s[r][3]) for r in range(G)]
            sq = [(s[r][0] * s[r][0] + s[r][1] * s[r][1])
                  + (s[r][2] * s[r][2] + s[r][3] * s[r][3]) for r in range(G)]
            for k in (8, 4, 2, 1):
                perm = lanes ^ np.int32(k)
                tot = [t + _lane_shuffle(t, perm) for t in tot]
                sq = [q + _lane_shuffle(q, perm) for q in sq]
            mean = [t * np.float32(1.0 / H) for t in tot]
            var = [q * np.float32(1.0 / H) - m * m for q, m in zip(sq, mean)]
            packed = var[0]
            for r in range(1, G):
                packed = jnp.where(lanes == np.int32(r), var[r], packed)
            rsq = _rsqrt_vec(packed + np.float32(EPS))
            rstd = [_lane_shuffle(rsq, lanes * 0 + np.int32(r))
                    for r in range(G)]
            for r in range(G):
                for j in range(4):
                    obufb[rb + r, pl.ds(16 * j, 16)] = (
                        (s[r][j] - mean[r]) * rstd[r] * ws[j] + bs[j])
            return 0

        lax.fori_loop(0, CHUNK // G, grp_body, 0)

    for b in (0, 1):
        pltpu.sync_copy(ids3_hbm.at[:, pl.ds(off_of(b), CHUNK)], slots[b][0])
        fire_gathers(slots[b])

    def pair_body(gg, _):
        for b in (0, 1):
            slot = slots[b]
            idxb, obufb, semi, semo = slot[0], slot[4], slot[6], slot[7]
            g = 2 * gg + b
            off_g = off_of(g)
            wait_gathers(slot)

            @pl.when(g + 2 < NCHUNKS)
            def _():
                pltpu.async_copy(
                    ids3_hbm.at[:, pl.ds(off_of(g + 2), CHUNK)], idxb, semi)

            @pl.when(g >= 2)
            def _():
                pltpu.make_async_copy(
                    obufb, out_hbm.at[pl.ds(0, CHUNK)], semo).wait()

            compute_chunk(slot)
            pltpu.async_copy(obufb, out_hbm.at[pl.ds(off_g, CHUNK)], semo)

            @pl.when(g + 2 < NCHUNKS)
            def _():
                pltpu.make_async_copy(
                    ids3_hbm.at[:, pl.ds(0, CHUNK)], idxb, semi).wait()
                fire_gathers(slot)

        return 0

    lax.fori_loop(0, NCHUNKS // 2, pair_body, 0)

    for b in (0, 1):
        pltpu.make_async_copy(
            slots[b][4], out_hbm.at[pl.ds(0, CHUNK)], slots[b][7]).wait()


@functools.partial(jax.jit, static_argnames=())
def _run(ids3, pos_tbl, hop_tbl, lnw, lnb):
    mesh = plsc.VectorSubcoreMesh(core_axis_name="c", subcore_axis_name="s",
                                  num_cores=NC, num_subcores=NS)
    f = pl.kernel(
        _body,
        out_type=jax.ShapeDtypeStruct((N, H), jnp.float32),
        mesh=mesh,
        compiler_params=pltpu.CompilerParams(needs_layout_passes=False,
                                             use_tc_tiling_on_sc=False),
        scratch_types=[
            pltpu.VMEM((3, CHUNK), jnp.int32),
            pltpu.VMEM((3, CHUNK), jnp.int32),
            pltpu.VMEM((CHUNK, H), jnp.float32),
            pltpu.VMEM((CHUNK, H), jnp.float32),
            pltpu.VMEM((CHUNK, H), jnp.float32),
            pltpu.VMEM((CHUNK, H), jnp.float32),
            pltpu.VMEM((CHUNK, H), jnp.float32),
            pltpu.VMEM((CHUNK, H), jnp.float32),
            pltpu.VMEM((CHUNK, H), jnp.float32),
            pltpu.VMEM((CHUNK, H), jnp.float32),
            pltpu.VMEM((H,), jnp.float32),
            pltpu.VMEM((H,), jnp.float32),
            pltpu.SemaphoreType.DMA,
            pltpu.SemaphoreType.DMA,
            pltpu.SemaphoreType.DMA,
            pltpu.SemaphoreType.DMA,
            pltpu.SemaphoreType.DMA,
            pltpu.SemaphoreType.DMA,
        ],
    )
    return f(ids3, pos_tbl, hop_tbl, lnw, lnb)


def kernel(init_pos_ids, hop_dis_ids, time_dis_ids, pos_table, hop_table,
           time_table, ln_weight, ln_bias):
    del time_table
    ids3 = jnp.stack([init_pos_ids.reshape(N).astype(jnp.int32),
                      hop_dis_ids.reshape(N).astype(jnp.int32),
                      time_dis_ids.reshape(N).astype(jnp.int32)])
    out = _run(ids3, pos_table, hop_table, ln_weight, ln_bias)
    return out.reshape(B, L, H)
